# baseline (device time: 8036 ns/iter reference)
import jax
import jax.numpy as jnp
from jax import lax
from jax.experimental import pallas as pl
from jax.experimental.pallas import tpu as pltpu

N_DEV = 8


def kernel(x):
    m_per, n = x.shape

    def body(x_ref, out_ref, x_vmem, out_vmem, comm_ref,
             send_sems, recv_sems, copy_sems):
        my = lax.axis_index("i")

        barrier_sem = pltpu.get_barrier_semaphore()
        for j in range(1, N_DEV):
            pl.semaphore_signal(
                barrier_sem, inc=1,
                device_id=((my + j) % N_DEV,),
                device_id_type=pl.DeviceIdType.MESH,
            )

        in_copy = pltpu.make_async_copy(x_ref, x_vmem, copy_sems.at[0])
        in_copy.start()
        in_copy.wait()
        part = jnp.max(x_vmem[...], axis=0, keepdims=True)

        pl.semaphore_wait(barrier_sem, N_DEV - 1)

        for src in range(N_DEV):
            @pl.when(my == src)
            def _(src=src):
                comm_ref[src] = part
                for dst in range(N_DEV):
                    if dst == src:
                        continue
                    pltpu.make_async_remote_copy(
                        src_ref=comm_ref.at[src],
                        dst_ref=comm_ref.at[src],
                        send_sem=send_sems.at[dst],
                        recv_sem=recv_sems.at[src],
                        device_id=(dst,),
                        device_id_type=pl.DeviceIdType.MESH,
                    ).start()

        for src in range(N_DEV):
            @pl.when(my != src)
            def _(src=src):
                pltpu.make_async_remote_copy(
                    src_ref=comm_ref.at[src],
                    dst_ref=comm_ref.at[src],
                    send_sem=send_sems.at[0],
                    recv_sem=recv_sems.at[src],
                    device_id=(0,),
                    device_id_type=pl.DeviceIdType.MESH,
                ).wait_recv()

        out_vmem[...] = jnp.max(comm_ref[...], axis=0)
        out_copy = pltpu.make_async_copy(out_vmem, out_ref, copy_sems.at[1])
        out_copy.start()

        for dst in range(N_DEV):
            @pl.when(my != dst)
            def _(dst=dst):
                pltpu.make_async_remote_copy(
                    src_ref=comm_ref.at[0],
                    dst_ref=comm_ref.at[0],
                    send_sem=send_sems.at[dst],
                    recv_sem=recv_sems.at[0],
                    device_id=(0,),
                    device_id_type=pl.DeviceIdType.MESH,
                ).wait_send()

        out_copy.wait()

    return pl.pallas_call(
        body,
        out_shape=jax.ShapeDtypeStruct((1, n), x.dtype),
        in_specs=[pl.BlockSpec(memory_space=pl.ANY)],
        out_specs=pl.BlockSpec(memory_space=pl.ANY),
        scratch_shapes=[
            pltpu.VMEM((m_per, n), x.dtype),
            pltpu.VMEM((1, n), x.dtype),
            pltpu.VMEM((N_DEV, 1, n), x.dtype),
            pltpu.SemaphoreType.DMA((N_DEV,)),
            pltpu.SemaphoreType.DMA((N_DEV,)),
            pltpu.SemaphoreType.DMA((2,)),
        ],
        compiler_params=pltpu.CompilerParams(collective_id=0),
    )(x)
